# Initial kernel scaffold; baseline (speedup 1.0000x reference)
import jax
import jax.numpy as jnp
from jax.experimental import pallas as pl


def kernel(lig_node_attr, rec_node_attr, lr_edge_attr, lr_edge_sh, W1_0, b1_0, W2_0, b2_0, Wtp_0, W1_1, b1_1, W2_1, b2_1, Wtp_1, lr_edge_index):
    raise NotImplementedError("write your pallas kernel here")


# speedup vs baseline: 2.2985x; 2.2985x over previous
import functools

import jax
import jax.numpy as jnp
from jax import lax
from jax.experimental import pallas as pl
from jax.experimental.pallas import tpu as pltpu
from jax.experimental.pallas import tpu_sc as plsc

NSF = 16
SHD = 9
OUTR = 28
OUTP = 32
NNODE = 50000
NCORE = 2
NSUB = 16
NW = NCORE * NSUB
CH = 7
BE = 2048

_f32 = jnp.float32


def _mesh():
    return plsc.VectorSubcoreMesh(core_axis_name="c", subcore_axis_name="s")


_SC_PARAMS = pltpu.CompilerParams(use_tc_tiling_on_sc=False)


def _sc_gather(table, idx2d):
    R = idx2d.shape[0]
    rpw = R // NW
    D = table.shape[1]

    @functools.partial(
        pl.kernel,
        out_type=jax.ShapeDtypeStruct((R * 128, D), _f32),
        mesh=_mesh(),
        scratch_types=[
            pltpu.VMEM((128,), jnp.int32),
            pltpu.VMEM((128, D), _f32),
            pltpu.SemaphoreType.DMA,
        ],
        compiler_params=_SC_PARAMS,
    )
    def gk(table_hbm, idx_hbm, out_hbm, idx_v, rows_v, sem):
        wid = lax.axis_index("s") * NCORE + lax.axis_index("c")

        @pl.loop(0, rpw)
        def _(g):
            r = wid * rpw + g
            pltpu.sync_copy(idx_hbm.at[r], idx_v)
            pltpu.async_copy(table_hbm.at[idx_v], rows_v, sem).wait()
            pltpu.sync_copy(rows_v, out_hbm.at[pl.ds(r * 128, 128)])

    return gk(table, idx2d)


def _sc_scatter_add(msg, idx2d, zrows):
    R = idx2d.shape[0]
    rpw = R // NW
    nchunks = rpw // CH
    rps = NNODE // NSUB

    @functools.partial(
        pl.kernel,
        out_type=jax.ShapeDtypeStruct((NCORE * NNODE, OUTP), _f32),
        mesh=_mesh(),
        scratch_types=[
            pltpu.VMEM((CH, 128), jnp.int32),
            pltpu.VMEM((CH * 128, OUTP), _f32),
            pltpu.VMEM_SHARED((NNODE, OUTP), _f32),
        ],
        compiler_params=_SC_PARAMS,
    )
    def sk(msg_hbm, idx_hbm, z_hbm, out_hbm, idx_v, msg_v, acc):
        cid = lax.axis_index("c")
        sid = lax.axis_index("s")
        wid = sid * NCORE + cid
        pltpu.sync_copy(z_hbm, acc.at[pl.ds(sid * rps, rps)])
        plsc.subcore_barrier()

        @pl.loop(0, nchunks)
        def _(t):
            r0 = wid * rpw + t * CH
            pltpu.sync_copy(idx_hbm.at[pl.ds(r0, CH)], idx_v)
            pltpu.sync_copy(msg_hbm.at[pl.ds(r0 * 128, CH * 128)], msg_v)
            for j in range(CH):
                pltpu.sync_copy(msg_v.at[pl.ds(j * 128, 128)],
                                acc.at[idx_v.at[j]], add=True)

        plsc.subcore_barrier()
        pltpu.sync_copy(acc.at[pl.ds(sid * rps, rps)],
                        out_hbm.at[pl.ds(cid * NNODE + sid * rps, rps)])

    return sk(msg, idx2d, zrows)


def _dot(a, b):
    return jax.lax.dot_general(a, b, (((1,), (0,)), ((), ())),
                               preferred_element_type=_f32)


def _tc_edge(ea, lg, rg, sh, w1a, w1b, w1c, b1, w2, b2, wtp9):
    EP = ea.shape[0]
    grid = (EP // BE,)

    def body(ea_r, lg_r, rg_r, sh_r, w1a_r, w1b_r, w1c_r, b1_r, w2_r, b2_r,
             wtp_r, out_r):
        rgv = rg_r[...]
        h = jnp.maximum(
            _dot(ea_r[...], w1a_r[...]) + _dot(lg_r[...], w1b_r[...])
            + _dot(rgv, w1c_r[...]) + b1_r[0:1, :], 0.0)
        gate = _dot(h, w2_r[...]) + b2_r[0:1, :]
        shv = sh_r[...]
        acc = _dot(rgv * shv[:, 0:1], wtp_r[0])
        for s in range(1, SHD):
            acc = acc + _dot(rgv * shv[:, s:s + 1], wtp_r[s])
        out_r[...] = acc * gate

    edge_spec = lambda d: pl.BlockSpec((BE, d), lambda i: (i, 0))
    full2 = lambda a, b: pl.BlockSpec((a, b), lambda i: (0, 0))

    return pl.pallas_call(
        body,
        grid=grid,
        in_specs=[
            edge_spec(NSF), edge_spec(NSF), edge_spec(NSF), edge_spec(SHD),
            full2(NSF, NSF), full2(NSF, NSF), full2(NSF, NSF),
            full2(8, NSF), full2(NSF, OUTP), full2(8, OUTP),
            pl.BlockSpec((SHD, NSF, OUTP), lambda i: (0, 0, 0)),
        ],
        out_specs=edge_spec(OUTP),
        out_shape=jax.ShapeDtypeStruct((EP, OUTP), _f32),
    )(ea, lg, rg, sh, w1a, w1b, w1c, b1, w2, b2, wtp9)


def _tc_combine(parts):
    BN = 2000
    nb = NNODE // BN

    def body(a_r, b_r, o16_r, o28_r):
        s = a_r[...] + b_r[...]
        o16_r[...] = s[:, :NSF]
        o28_r[...] = s[:, :OUTR]

    return pl.pallas_call(
        body,
        grid=(nb,),
        in_specs=[
            pl.BlockSpec((BN, OUTP), lambda i: (i, 0)),
            pl.BlockSpec((BN, OUTP), lambda i: (i + nb, 0)),
        ],
        out_specs=[
            pl.BlockSpec((BN, NSF), lambda i: (i, 0)),
            pl.BlockSpec((BN, OUTR), lambda i: (i, 0)),
        ],
        out_shape=[
            jax.ShapeDtypeStruct((NNODE, NSF), _f32),
            jax.ShapeDtypeStruct((NNODE, OUTR), _f32),
        ],
    )(parts, parts)


def _prep_params(W1, b1, W2, b2, Wtp):
    w1a, w1b, w1c = W1[:NSF], W1[NSF:2 * NSF], W1[2 * NSF:]
    b1b = jnp.broadcast_to(b1[None, :], (8, NSF))
    w2p = jnp.pad(W2, ((0, 0), (0, OUTP - OUTR)))
    b2b = jnp.broadcast_to(jnp.pad(b2, (0, OUTP - OUTR))[None, :], (8, OUTP))
    wtp9 = jnp.pad(Wtp, ((0, 0), (0, OUTP - OUTR)))
    wtp9 = wtp9.reshape(NSF, SHD, OUTP).transpose(1, 0, 2)
    return w1a, w1b, w1c, b1b, w2p, b2b, wtp9


def kernel(lig_node_attr, rec_node_attr, lr_edge_attr, lr_edge_sh,
           W1_0, b1_0, W2_0, b2_0, Wtp_0,
           W1_1, b1_1, W2_1, b2_1, Wtp_1,
           lr_edge_index):
    E = lr_edge_attr.shape[0]
    step = NW * CH * 128
    EP = ((E + step - 1) // step) * step
    pad = EP - E

    src = lr_edge_index[0].astype(jnp.int32)
    dst = lr_edge_index[1].astype(jnp.int32)
    src2d = jnp.concatenate([src, jnp.zeros((pad,), jnp.int32)]
                            ).reshape(EP // 128, 128)
    dst2d = jnp.concatenate([dst, jnp.zeros((pad,), jnp.int32)]
                            ).reshape(EP // 128, 128)
    ea_p = jnp.concatenate([lr_edge_attr, jnp.zeros((pad, NSF), _f32)])
    sh_p = jnp.concatenate([lr_edge_sh, jnp.zeros((pad, SHD), _f32)])
    zrows = jnp.zeros((NNODE // NSUB, OUTP), _f32)

    rec_g = _sc_gather(rec_node_attr, dst2d)
    lig_g = _sc_gather(lig_node_attr, src2d)

    msg0 = _tc_edge(ea_p, lig_g, rec_g, sh_p,
                    *_prep_params(W1_0, b1_0, W2_0, b2_0, Wtp_0))
    parts0 = _sc_scatter_add(msg0, src2d, zrows)
    out0_16, _ = _tc_combine(parts0)

    lig_g1 = _sc_gather(out0_16, src2d)
    msg1 = _tc_edge(ea_p, lig_g1, rec_g, sh_p,
                    *_prep_params(W1_1, b1_1, W2_1, b2_1, Wtp_1))
    parts1 = _sc_scatter_add(msg1, src2d, zrows)
    _, out = _tc_combine(parts1)
    return out


# speedup vs baseline: 2.4760x; 1.0772x over previous
import functools

import jax
import jax.numpy as jnp
from jax import lax
from jax.experimental import pallas as pl
from jax.experimental.pallas import tpu as pltpu
from jax.experimental.pallas import tpu_sc as plsc

NSF = 16
SHD = 9
OUTR = 28
OUTP = 32
NNODE = 50000
NCORE = 2
NSUB = 16
NW = NCORE * NSUB
CH = 7
BE = 2048

_f32 = jnp.float32


def _mesh():
    return plsc.VectorSubcoreMesh(core_axis_name="c", subcore_axis_name="s")


_SC_PARAMS = pltpu.CompilerParams(use_tc_tiling_on_sc=False)


def _sc_gather(tables, idx2ds):
    npair = len(tables)
    R = idx2ds[0].shape[0]
    rpw = R // NW
    ngrp = rpw // CH
    half = ngrp // 2
    D = tables[0].shape[1]

    scratch = ([pltpu.VMEM((rpw, 128), jnp.int32) for _ in range(npair)]
               + [pltpu.VMEM((CH * 128, D), _f32) for _ in range(2 * npair)]
               + [pltpu.SemaphoreType.DMA, pltpu.SemaphoreType.DMA,
                  pltpu.SemaphoreType.DMA])

    @functools.partial(
        pl.kernel,
        out_type=[jax.ShapeDtypeStruct((R * 128, D), _f32)
                  for _ in range(npair)],
        mesh=_mesh(),
        scratch_types=scratch,
        compiler_params=_SC_PARAMS,
    )
    def gk(*refs):
        table_h = refs[:npair]
        idx_h = refs[npair:2 * npair]
        out_h = refs[2 * npair:3 * npair]
        idx_v = refs[3 * npair:4 * npair]
        bank = refs[4 * npair:6 * npair]
        gsem, wsem0, wsem1 = refs[6 * npair:]
        wid = lax.axis_index("s") * NCORE + lax.axis_index("c")
        base = wid * rpw

        for p in range(npair):
            pltpu.sync_copy(idx_h[p].at[pl.ds(base, rpw)], idx_v[p])

        def fire_gather(g, b):
            for p in range(npair):
                for j in range(CH):
                    pltpu.async_copy(
                        table_h[p].at[idx_v[p].at[g * CH + j]],
                        bank[b * npair + p].at[pl.ds(j * 128, 128)], gsem)

        def wait_gather():
            for p in range(npair):
                for j in range(CH):
                    pltpu.make_async_copy(
                        table_h[p].at[idx_v[p].at[j]],
                        bank[p].at[pl.ds(j * 128, 128)], gsem).wait()

        def fire_wb(g, b, sem):
            for p in range(npair):
                pltpu.async_copy(
                    bank[b * npair + p],
                    out_h[p].at[pl.ds((base + g * CH) * 128, CH * 128)], sem)

        def wait_wb(sem):
            for p in range(npair):
                pltpu.make_async_copy(
                    bank[p], out_h[p].at[pl.ds(base * 128, CH * 128)],
                    sem).wait()

        fire_gather(0, 0)

        @pl.loop(0, half)
        def _(t):
            wait_gather()
            fire_wb(2 * t, 0, wsem0)

            @pl.when(t > 0)
            def _():
                wait_wb(wsem1)
            fire_gather(2 * t + 1, 1)
            wait_gather()
            wait_wb(wsem0)

            @pl.when(t < half - 1)
            def _():
                fire_gather(2 * t + 2, 0)
            fire_wb(2 * t + 1, 1, wsem1)

        wait_wb(wsem1)

    outs = gk(*tables, *idx2ds)
    return list(outs) if isinstance(outs, (list, tuple)) else [outs]


def _sc_scatter_add(msg, idx2d, zrows):
    R = idx2d.shape[0]
    rpw = R // NW
    SCH = 2
    ngrp = rpw // SCH
    half = ngrp // 2
    rps = NNODE // NSUB

    @functools.partial(
        pl.kernel,
        out_type=jax.ShapeDtypeStruct((NCORE * NNODE, OUTP), _f32),
        mesh=_mesh(),
        scratch_types=[
            pltpu.VMEM((SCH, 128), jnp.int32),
            pltpu.VMEM((SCH, 128), jnp.int32),
            pltpu.VMEM((SCH * 128, OUTP), _f32),
            pltpu.VMEM((SCH * 128, OUTP), _f32),
            pltpu.VMEM_SHARED((NNODE, OUTP), _f32),
            pltpu.SemaphoreType.DMA,
            pltpu.SemaphoreType.DMA,
            pltpu.SemaphoreType.DMA,
        ],
        compiler_params=_SC_PARAMS,
    )
    def sk(msg_hbm, idx_hbm, z_hbm, out_hbm, ib0, ib1, mb0, mb1, acc,
           lsem0, lsem1, ssem):
        cid = lax.axis_index("c")
        sid = lax.axis_index("s")
        wid = sid * NCORE + cid
        base = wid * rpw
        pltpu.sync_copy(z_hbm, acc.at[pl.ds(sid * rps, rps)])
        plsc.subcore_barrier()

        def fire_load(g, ib, mb, sem):
            pltpu.async_copy(idx_hbm.at[pl.ds(base + g * SCH, SCH)], ib, sem)
            pltpu.async_copy(
                msg_hbm.at[pl.ds((base + g * SCH) * 128, SCH * 128)], mb, sem)

        def wait_load(ib, mb, sem):
            pltpu.make_async_copy(
                idx_hbm.at[pl.ds(base, SCH)], ib, sem).wait()
            pltpu.make_async_copy(
                msg_hbm.at[pl.ds(base * 128, SCH * 128)], mb, sem).wait()

        def scatter_group(ib, mb):
            descs = []
            for j in range(SCH):
                descs.append(pltpu.async_copy(
                    mb.at[pl.ds(j * 128, 128)],
                    acc.at[ib.at[j]], ssem, add=True))
            for d in descs:
                d.wait()

        fire_load(0, ib0, mb0, lsem0)

        @pl.loop(0, half)
        def _(t):
            wait_load(ib0, mb0, lsem0)
            fire_load(2 * t + 1, ib1, mb1, lsem1)
            scatter_group(ib0, mb0)
            wait_load(ib1, mb1, lsem1)

            @pl.when(t < half - 1)
            def _():
                fire_load(2 * t + 2, ib0, mb0, lsem0)
            scatter_group(ib1, mb1)

        plsc.subcore_barrier()
        pltpu.sync_copy(acc.at[pl.ds(sid * rps, rps)],
                        out_hbm.at[pl.ds(cid * NNODE + sid * rps, rps)])

    return sk(msg, idx2d, zrows)


def _dot(a, b):
    return jax.lax.dot_general(a, b, (((1,), (0,)), ((), ())),
                               preferred_element_type=_f32)


def _tc_edge(ea, lg, rg, sh, w1a, w1b, w1c, b1, w2, b2, wtp9):
    EP = ea.shape[0]
    grid = (EP // BE,)

    def body(ea_r, lg_r, rg_r, sh_r, w1a_r, w1b_r, w1c_r, b1_r, w2_r, b2_r,
             wtp_r, out_r):
        rgv = rg_r[...]
        h = jnp.maximum(
            _dot(ea_r[...], w1a_r[...]) + _dot(lg_r[...], w1b_r[...])
            + _dot(rgv, w1c_r[...]) + b1_r[0:1, :], 0.0)
        gate = _dot(h, w2_r[...]) + b2_r[0:1, :]
        shv = sh_r[...]
        acc = _dot(rgv * shv[:, 0:1], wtp_r[0])
        for s in range(1, SHD):
            acc = acc + _dot(rgv * shv[:, s:s + 1], wtp_r[s])
        out_r[...] = acc * gate

    edge_spec = lambda d: pl.BlockSpec((BE, d), lambda i: (i, 0))
    full2 = lambda a, b: pl.BlockSpec((a, b), lambda i: (0, 0))

    return pl.pallas_call(
        body,
        grid=grid,
        in_specs=[
            edge_spec(NSF), edge_spec(NSF), edge_spec(NSF), edge_spec(SHD),
            full2(NSF, NSF), full2(NSF, NSF), full2(NSF, NSF),
            full2(8, NSF), full2(NSF, OUTP), full2(8, OUTP),
            pl.BlockSpec((SHD, NSF, OUTP), lambda i: (0, 0, 0)),
        ],
        out_specs=edge_spec(OUTP),
        out_shape=jax.ShapeDtypeStruct((EP, OUTP), _f32),
    )(ea, lg, rg, sh, w1a, w1b, w1c, b1, w2, b2, wtp9)


def _tc_combine(parts):
    BN = 2000
    nb = NNODE // BN

    def body(a_r, b_r, o16_r, o28_r):
        s = a_r[...] + b_r[...]
        o16_r[...] = s[:, :NSF]
        o28_r[...] = s[:, :OUTR]

    return pl.pallas_call(
        body,
        grid=(nb,),
        in_specs=[
            pl.BlockSpec((BN, OUTP), lambda i: (i, 0)),
            pl.BlockSpec((BN, OUTP), lambda i: (i + nb, 0)),
        ],
        out_specs=[
            pl.BlockSpec((BN, NSF), lambda i: (i, 0)),
            pl.BlockSpec((BN, OUTR), lambda i: (i, 0)),
        ],
        out_shape=[
            jax.ShapeDtypeStruct((NNODE, NSF), _f32),
            jax.ShapeDtypeStruct((NNODE, OUTR), _f32),
        ],
    )(parts, parts)


def _prep_params(W1, b1, W2, b2, Wtp):
    w1a, w1b, w1c = W1[:NSF], W1[NSF:2 * NSF], W1[2 * NSF:]
    b1b = jnp.broadcast_to(b1[None, :], (8, NSF))
    w2p = jnp.pad(W2, ((0, 0), (0, OUTP - OUTR)))
    b2b = jnp.broadcast_to(jnp.pad(b2, (0, OUTP - OUTR))[None, :], (8, OUTP))
    wtp9 = jnp.pad(Wtp, ((0, 0), (0, OUTP - OUTR)))
    wtp9 = wtp9.reshape(NSF, SHD, OUTP).transpose(1, 0, 2)
    return w1a, w1b, w1c, b1b, w2p, b2b, wtp9


def kernel(lig_node_attr, rec_node_attr, lr_edge_attr, lr_edge_sh,
           W1_0, b1_0, W2_0, b2_0, Wtp_0,
           W1_1, b1_1, W2_1, b2_1, Wtp_1,
           lr_edge_index):
    E = lr_edge_attr.shape[0]
    step = NW * CH * 128
    EP = ((E + step - 1) // step) * step
    pad = EP - E

    src = lr_edge_index[0].astype(jnp.int32)
    dst = lr_edge_index[1].astype(jnp.int32)
    src2d = jnp.concatenate([src, jnp.zeros((pad,), jnp.int32)]
                            ).reshape(EP // 128, 128)
    dst2d = jnp.concatenate([dst, jnp.zeros((pad,), jnp.int32)]
                            ).reshape(EP // 128, 128)
    ea_p = jnp.concatenate([lr_edge_attr, jnp.zeros((pad, NSF), _f32)])
    sh_p = jnp.concatenate([lr_edge_sh, jnp.zeros((pad, SHD), _f32)])
    zrows = jnp.zeros((NNODE // NSUB, OUTP), _f32)

    rec_g, lig_g = _sc_gather([rec_node_attr, lig_node_attr], [dst2d, src2d])

    msg0 = _tc_edge(ea_p, lig_g, rec_g, sh_p,
                    *_prep_params(W1_0, b1_0, W2_0, b2_0, Wtp_0))
    parts0 = _sc_scatter_add(msg0, src2d, zrows)
    out0_16, _ = _tc_combine(parts0)

    lig_g1, = _sc_gather([out0_16], [src2d])
    msg1 = _tc_edge(ea_p, lig_g1, rec_g, sh_p,
                    *_prep_params(W1_1, b1_1, W2_1, b2_1, Wtp_1))
    parts1 = _sc_scatter_add(msg1, src2d, zrows)
    _, out = _tc_combine(parts1)
    return out


# speedup vs baseline: 2.6586x; 1.0737x over previous
import functools

import jax
import jax.numpy as jnp
from jax import lax
from jax.experimental import pallas as pl
from jax.experimental.pallas import tpu as pltpu
from jax.experimental.pallas import tpu_sc as plsc

NSF = 16
SHD = 9
OUTR = 28
OUTP = 32
NNODE = 50000
NCORE = 2
NSUB = 16
NW = NCORE * NSUB
CH = 7
BE = 2048

_f32 = jnp.float32


def _mesh():
    return plsc.VectorSubcoreMesh(core_axis_name="c", subcore_axis_name="s")


_SC_PARAMS = pltpu.CompilerParams(use_tc_tiling_on_sc=False)


def _sc_gather(tables, idx2ds):
    npair = len(tables)
    R = idx2ds[0].shape[0]
    rpw = R // NW
    ngrp = rpw // CH
    half = ngrp // 2
    D = tables[0].shape[1]

    scratch = ([pltpu.VMEM((rpw, 128), jnp.int32) for _ in range(npair)]
               + [pltpu.VMEM((CH * 128, D), _f32) for _ in range(2 * npair)]
               + [pltpu.SemaphoreType.DMA, pltpu.SemaphoreType.DMA,
                  pltpu.SemaphoreType.DMA])

    @functools.partial(
        pl.kernel,
        out_type=[jax.ShapeDtypeStruct((R * 128, D), _f32)
                  for _ in range(npair)],
        mesh=_mesh(),
        scratch_types=scratch,
        compiler_params=_SC_PARAMS,
    )
    def gk(*refs):
        table_h = refs[:npair]
        idx_h = refs[npair:2 * npair]
        out_h = refs[2 * npair:3 * npair]
        idx_v = refs[3 * npair:4 * npair]
        bank = refs[4 * npair:6 * npair]
        gsem, wsem0, wsem1 = refs[6 * npair:]
        wid = lax.axis_index("s") * NCORE + lax.axis_index("c")
        base = wid * rpw

        for p in range(npair):
            pltpu.sync_copy(idx_h[p].at[pl.ds(base, rpw)], idx_v[p])

        def fire_gather(g, b):
            for p in range(npair):
                for j in range(CH):
                    pltpu.async_copy(
                        table_h[p].at[idx_v[p].at[g * CH + j]],
                        bank[b * npair + p].at[pl.ds(j * 128, 128)], gsem)

        def wait_gather():
            for p in range(npair):
                for j in range(CH):
                    pltpu.make_async_copy(
                        table_h[p].at[idx_v[p].at[j]],
                        bank[p].at[pl.ds(j * 128, 128)], gsem).wait()

        def fire_wb(g, b, sem):
            for p in range(npair):
                pltpu.async_copy(
                    bank[b * npair + p],
                    out_h[p].at[pl.ds((base + g * CH) * 128, CH * 128)], sem)

        def wait_wb(sem):
            for p in range(npair):
                pltpu.make_async_copy(
                    bank[p], out_h[p].at[pl.ds(base * 128, CH * 128)],
                    sem).wait()

        fire_gather(0, 0)

        @pl.loop(0, half)
        def _(t):
            wait_gather()
            fire_wb(2 * t, 0, wsem0)

            @pl.when(t > 0)
            def _():
                wait_wb(wsem1)
            fire_gather(2 * t + 1, 1)
            wait_gather()
            wait_wb(wsem0)

            @pl.when(t < half - 1)
            def _():
                fire_gather(2 * t + 2, 0)
            fire_wb(2 * t + 1, 1, wsem1)

        wait_wb(wsem1)

    outs = gk(*tables, *idx2ds)
    return list(outs) if isinstance(outs, (list, tuple)) else [outs]


def _sc_scatter_add(msg, idx2d, zrows):
    R = idx2d.shape[0]
    rpw = R // NW
    SCH = 2
    ngrp = rpw // SCH
    half = ngrp // 2
    rps = NNODE // NSUB

    @functools.partial(
        pl.kernel,
        out_type=jax.ShapeDtypeStruct((NCORE * NNODE, OUTP), _f32),
        mesh=_mesh(),
        scratch_types=[
            pltpu.VMEM((SCH, 128), jnp.int32),
            pltpu.VMEM((SCH, 128), jnp.int32),
            pltpu.VMEM((SCH * 128, OUTP), _f32),
            pltpu.VMEM((SCH * 128, OUTP), _f32),
            pltpu.VMEM_SHARED((NNODE, OUTP), _f32),
            pltpu.SemaphoreType.DMA,
            pltpu.SemaphoreType.DMA,
            pltpu.SemaphoreType.DMA,
        ],
        compiler_params=_SC_PARAMS,
    )
    def sk(msg_hbm, idx_hbm, z_hbm, out_hbm, ib0, ib1, mb0, mb1, acc,
           lsem0, lsem1, ssem):
        cid = lax.axis_index("c")
        sid = lax.axis_index("s")
        wid = sid * NCORE + cid
        base = wid * rpw
        pltpu.sync_copy(z_hbm, acc.at[pl.ds(sid * rps, rps)])
        plsc.subcore_barrier()

        def fire_load(g, ib, mb, sem):
            pltpu.async_copy(idx_hbm.at[pl.ds(base + g * SCH, SCH)], ib, sem)
            pltpu.async_copy(
                msg_hbm.at[pl.ds((base + g * SCH) * 128, SCH * 128)], mb, sem)

        def wait_load(ib, mb, sem):
            pltpu.make_async_copy(
                idx_hbm.at[pl.ds(base, SCH)], ib, sem).wait()
            pltpu.make_async_copy(
                msg_hbm.at[pl.ds(base * 128, SCH * 128)], mb, sem).wait()

        def scatter_group(ib, mb):
            descs = []
            for j in range(SCH):
                descs.append(pltpu.async_copy(
                    mb.at[pl.ds(j * 128, 128)],
                    acc.at[ib.at[j]], ssem, add=True))
            for d in descs:
                d.wait()

        fire_load(0, ib0, mb0, lsem0)

        @pl.loop(0, half)
        def _(t):
            wait_load(ib0, mb0, lsem0)
            fire_load(2 * t + 1, ib1, mb1, lsem1)
            scatter_group(ib0, mb0)
            wait_load(ib1, mb1, lsem1)

            @pl.when(t < half - 1)
            def _():
                fire_load(2 * t + 2, ib0, mb0, lsem0)
            scatter_group(ib1, mb1)

        plsc.subcore_barrier()
        pltpu.sync_copy(acc.at[pl.ds(sid * rps, rps)],
                        out_hbm.at[pl.ds(cid * NNODE + sid * rps, rps)])

    return sk(msg, idx2d, zrows)


def _dot(a, b):
    return jax.lax.dot_general(a, b, (((1,), (0,)), ((), ())),
                               preferred_element_type=_f32)


def _tc_edge(ea, lg, rg, sh, w1a, w1b, w1c, b1, w2, b2, wtpall, repmat,
             summat, EP):
    E = ea.shape[0]
    grid = (EP // BE,)
    last = (E - 1) // BE
    RW = SHD * OUTP

    def body(ea_r, lg_r, rg_r, sh_r, w1a_r, w1b_r, w1c_r, b1_r, w2_r, b2_r,
             wtp_r, rep_r, sum_r, out_r):
        i = pl.program_id(0)
        rgv = rg_r[...]
        h = jnp.maximum(
            _dot(ea_r[...], w1a_r[...]) + _dot(lg_r[...], w1b_r[...])
            + _dot(rgv, w1c_r[...]) + b1_r[0:1, :], 0.0)
        gate = _dot(h, w2_r[...]) + b2_r[0:1, :]
        u = _dot(rgv, wtp_r[...])
        s = _dot(sh_r[...], rep_r[...])
        msg = _dot(u * s, sum_r[...]) * gate
        out_r[...] = msg

        @pl.when(i >= last)
        def _():
            rows = lax.broadcasted_iota(jnp.int32, (BE, 1), 0) + i * BE
            out_r[...] = jnp.where(rows < E, msg, 0.0)

    clamp_spec = lambda d: pl.BlockSpec(
        (BE, d), lambda i: (jnp.minimum(i, last), 0))
    edge_spec = lambda d: pl.BlockSpec((BE, d), lambda i: (i, 0))
    full2 = lambda a, b: pl.BlockSpec((a, b), lambda i: (0, 0))

    return pl.pallas_call(
        body,
        grid=grid,
        in_specs=[
            clamp_spec(NSF), edge_spec(NSF), edge_spec(NSF), clamp_spec(SHD),
            full2(NSF, NSF), full2(NSF, NSF), full2(NSF, NSF),
            full2(8, NSF), full2(NSF, OUTP), full2(8, OUTP),
            full2(NSF, RW), full2(SHD, RW), full2(RW, OUTP),
        ],
        out_specs=edge_spec(OUTP),
        out_shape=jax.ShapeDtypeStruct((EP, OUTP), _f32),
    )(ea, lg, rg, sh, w1a, w1b, w1c, b1, w2, b2, wtpall, repmat, summat)


def _tc_combine(parts):
    BN = 2000
    nb = NNODE // BN

    def body(a_r, b_r, o16_r, o28_r):
        s = a_r[...] + b_r[...]
        o16_r[...] = s[:, :NSF]
        o28_r[...] = s[:, :OUTR]

    return pl.pallas_call(
        body,
        grid=(nb,),
        in_specs=[
            pl.BlockSpec((BN, OUTP), lambda i: (i, 0)),
            pl.BlockSpec((BN, OUTP), lambda i: (i + nb, 0)),
        ],
        out_specs=[
            pl.BlockSpec((BN, NSF), lambda i: (i, 0)),
            pl.BlockSpec((BN, OUTR), lambda i: (i, 0)),
        ],
        out_shape=[
            jax.ShapeDtypeStruct((NNODE, NSF), _f32),
            jax.ShapeDtypeStruct((NNODE, OUTR), _f32),
        ],
    )(parts, parts)


def _prep_params(W1, b1, W2, b2, Wtp):
    w1a, w1b, w1c = W1[:NSF], W1[NSF:2 * NSF], W1[2 * NSF:]
    b1b = jnp.broadcast_to(b1[None, :], (8, NSF))
    w2p = jnp.pad(W2, ((0, 0), (0, OUTP - OUTR)))
    b2b = jnp.broadcast_to(jnp.pad(b2, (0, OUTP - OUTR))[None, :], (8, OUTP))
    wtpall = jnp.pad(Wtp, ((0, 0), (0, OUTP - OUTR))
                     ).reshape(NSF, SHD * OUTP)
    return w1a, w1b, w1c, b1b, w2p, b2b, wtpall


def _rep_sum_mats():
    rw = SHD * OUTP
    repmat = (jnp.arange(rw)[None, :] // OUTP
              == jnp.arange(SHD)[:, None]).astype(_f32)
    summat = (jnp.arange(rw)[:, None] % OUTP
              == jnp.arange(OUTP)[None, :]).astype(_f32)
    return repmat, summat


def kernel(lig_node_attr, rec_node_attr, lr_edge_attr, lr_edge_sh,
           W1_0, b1_0, W2_0, b2_0, Wtp_0,
           W1_1, b1_1, W2_1, b2_1, Wtp_1,
           lr_edge_index):
    E = lr_edge_attr.shape[0]
    step = NW * CH * 128
    EP = ((E + step - 1) // step) * step
    pad = EP - E

    src = lr_edge_index[0].astype(jnp.int32)
    dst = lr_edge_index[1].astype(jnp.int32)
    src2d = jnp.concatenate([src, jnp.zeros((pad,), jnp.int32)]
                            ).reshape(EP // 128, 128)
    dst2d = jnp.concatenate([dst, jnp.zeros((pad,), jnp.int32)]
                            ).reshape(EP // 128, 128)
    zrows = jnp.zeros((NNODE // NSUB, OUTP), _f32)
    repmat, summat = _rep_sum_mats()

    rec_g, lig_g = _sc_gather([rec_node_attr, lig_node_attr], [dst2d, src2d])

    msg0 = _tc_edge(lr_edge_attr, lig_g, rec_g, lr_edge_sh,
                    *_prep_params(W1_0, b1_0, W2_0, b2_0, Wtp_0),
                    repmat, summat, EP)
    parts0 = _sc_scatter_add(msg0, src2d, zrows)
    out0_16, _ = _tc_combine(parts0)

    lig_g1, = _sc_gather([out0_16], [src2d])
    msg1 = _tc_edge(lr_edge_attr, lig_g1, rec_g, lr_edge_sh,
                    *_prep_params(W1_1, b1_1, W2_1, b2_1, Wtp_1),
                    repmat, summat, EP)
    parts1 = _sc_scatter_add(msg1, src2d, zrows)
    _, out = _tc_combine(parts1)
    return out


# speedup vs baseline: 3.2592x; 1.2259x over previous
import functools

import jax
import jax.numpy as jnp
from jax import lax
from jax.experimental import pallas as pl
from jax.experimental.pallas import tpu as pltpu
from jax.experimental.pallas import tpu_sc as plsc

NSF = 16
SHD = 9
OUTR = 28
OUTP = 32
NNODE = 50000
NCORE = 2
NSUB = 16
NW = NCORE * NSUB
CH = 7
BE = 2048

_f32 = jnp.float32


def _mesh():
    return plsc.VectorSubcoreMesh(core_axis_name="c", subcore_axis_name="s")


_SC_PARAMS = pltpu.CompilerParams(use_tc_tiling_on_sc=False)


def _sc_gather(tables, idx2ds):
    npair = len(tables)
    R = idx2ds[0].shape[0]
    rpw = R // NW
    ngrp = rpw // CH
    half = ngrp // 2
    D = tables[0].shape[1]

    scratch = ([pltpu.VMEM((rpw, 128), jnp.int32) for _ in range(npair)]
               + [pltpu.VMEM((CH * 128, D), _f32) for _ in range(2 * npair)]
               + [pltpu.SemaphoreType.DMA, pltpu.SemaphoreType.DMA,
                  pltpu.SemaphoreType.DMA])

    @functools.partial(
        pl.kernel,
        out_type=[jax.ShapeDtypeStruct((R * 128, D), _f32)
                  for _ in range(npair)],
        mesh=_mesh(),
        scratch_types=scratch,
        compiler_params=_SC_PARAMS,
    )
    def gk(*refs):
        table_h = refs[:npair]
        idx_h = refs[npair:2 * npair]
        out_h = refs[2 * npair:3 * npair]
        idx_v = refs[3 * npair:4 * npair]
        bank = refs[4 * npair:6 * npair]
        gsem, wsem0, wsem1 = refs[6 * npair:]
        wid = lax.axis_index("s") * NCORE + lax.axis_index("c")
        base = wid * rpw

        for p in range(npair):
            pltpu.sync_copy(idx_h[p].at[pl.ds(base, rpw)], idx_v[p])

        def fire_gather(g, b):
            for p in range(npair):
                for j in range(CH):
                    pltpu.async_copy(
                        table_h[p].at[idx_v[p].at[g * CH + j]],
                        bank[b * npair + p].at[pl.ds(j * 128, 128)], gsem)

        def wait_gather():
            for p in range(npair):
                for j in range(CH):
                    pltpu.make_async_copy(
                        table_h[p].at[idx_v[p].at[j]],
                        bank[p].at[pl.ds(j * 128, 128)], gsem).wait()

        def fire_wb(g, b, sem):
            for p in range(npair):
                pltpu.async_copy(
                    bank[b * npair + p],
                    out_h[p].at[pl.ds((base + g * CH) * 128, CH * 128)], sem)

        def wait_wb(sem):
            for p in range(npair):
                pltpu.make_async_copy(
                    bank[p], out_h[p].at[pl.ds(base * 128, CH * 128)],
                    sem).wait()

        fire_gather(0, 0)

        @pl.loop(0, half)
        def _(t):
            wait_gather()
            fire_wb(2 * t, 0, wsem0)

            @pl.when(t > 0)
            def _():
                wait_wb(wsem1)
            fire_gather(2 * t + 1, 1)
            wait_gather()
            wait_wb(wsem0)

            @pl.when(t < half - 1)
            def _():
                fire_gather(2 * t + 2, 0)
            fire_wb(2 * t + 1, 1, wsem1)

        wait_wb(wsem1)

    outs = gk(*tables, *idx2ds)
    return list(outs) if isinstance(outs, (list, tuple)) else [outs]


def _sc_scatter_add(msg, idx2d, zrows):
    R = idx2d.shape[0]
    rpw = R // NW
    SCH = 2
    ngrp = rpw // SCH
    half = ngrp // 2
    rps = NNODE // NSUB

    @functools.partial(
        pl.kernel,
        out_type=jax.ShapeDtypeStruct((NCORE * NNODE, OUTP), _f32),
        mesh=_mesh(),
        scratch_types=[
            pltpu.VMEM((SCH, 128), jnp.int32),
            pltpu.VMEM((SCH, 128), jnp.int32),
            pltpu.VMEM((SCH * 128, OUTP), _f32),
            pltpu.VMEM((SCH * 128, OUTP), _f32),
            pltpu.VMEM_SHARED((NNODE, OUTP), _f32),
            pltpu.SemaphoreType.DMA,
            pltpu.SemaphoreType.DMA,
            pltpu.SemaphoreType.DMA,
        ],
        compiler_params=_SC_PARAMS,
    )
    def sk(msg_hbm, idx_hbm, z_hbm, out_hbm, ib0, ib1, mb0, mb1, acc,
           lsem0, lsem1, ssem):
        cid = lax.axis_index("c")
        sid = lax.axis_index("s")
        wid = sid * NCORE + cid
        base = wid * rpw
        pltpu.sync_copy(z_hbm, acc.at[pl.ds(sid * rps, rps)])
        plsc.subcore_barrier()

        def fire_load(g, ib, mb, sem):
            pltpu.async_copy(idx_hbm.at[pl.ds(base + g * SCH, SCH)], ib, sem)
            pltpu.async_copy(
                msg_hbm.at[pl.ds((base + g * SCH) * 128, SCH * 128)], mb, sem)

        def wait_load(ib, mb, sem):
            pltpu.make_async_copy(
                idx_hbm.at[pl.ds(base, SCH)], ib, sem).wait()
            pltpu.make_async_copy(
                msg_hbm.at[pl.ds(base * 128, SCH * 128)], mb, sem).wait()

        def scatter_group(ib, mb):
            descs = []
            for j in range(SCH):
                descs.append(pltpu.async_copy(
                    mb.at[pl.ds(j * 128, 128)],
                    acc.at[ib.at[j]], ssem, add=True))
            for d in descs:
                d.wait()

        fire_load(0, ib0, mb0, lsem0)

        @pl.loop(0, half)
        def _(t):
            wait_load(ib0, mb0, lsem0)
            fire_load(2 * t + 1, ib1, mb1, lsem1)
            scatter_group(ib0, mb0)
            wait_load(ib1, mb1, lsem1)

            @pl.when(t < half - 1)
            def _():
                fire_load(2 * t + 2, ib0, mb0, lsem0)
            scatter_group(ib1, mb1)

        plsc.subcore_barrier()
        pltpu.sync_copy(acc.at[pl.ds(sid * rps, rps)],
                        out_hbm.at[pl.ds(cid * NNODE + sid * rps, rps)])

    return sk(msg, idx2d, zrows)


def _dot(a, b):
    return jax.lax.dot_general(a, b, (((1,), (0,)), ((), ())),
                               preferred_element_type=_f32)


def _dot_c0(a, b):
    return jax.lax.dot_general(a, b, (((0,), (0,)), ((), ())),
                               preferred_element_type=_f32)


def _tc_edge(eaP, lgP, rgP, shP, w1a, w1b, w1c, b1, w2, b2, wtpall, repmat,
             summat):
    E = eaP.shape[0] * 8
    RP = lgP.shape[0]
    BR = BE // 8
    n = RP // BR
    lastE = (E // 8 - 1) // BR
    RW = SHD * OUTR

    def unpack(p):
        return jnp.concatenate([p[:, 16 * j:16 * (j + 1)] for j in range(8)],
                               axis=0)

    def repack(m):
        return jnp.concatenate([m[j * BR:(j + 1) * BR, :] for j in range(8)],
                               axis=1)

    def body(ea_r, lg_r, rg_r, sh_r, w1a_r, w1b_r, w1c_r, b1_r, w2_r,
             b2_r, wtp_r, rep_r, sum_r, out_r):
        i = pl.program_id(0)
        eav = unpack(ea_r[...])
        lgv = unpack(lg_r[...])
        rgv = unpack(rg_r[...])
        shv = unpack(sh_r[...])
        h = jnp.maximum(
            _dot(eav, w1a_r[...]) + _dot(lgv, w1b_r[...])
            + _dot(rgv, w1c_r[...]) + b1_r[0:1, :], 0.0)
        gate = _dot(h, w2_r[...]) + b2_r[0:1, :]
        u = _dot(rgv, wtp_r[...])
        s = _dot(shv, rep_r[...])
        msg = _dot(u * s, sum_r[...]) * gate

        @pl.when(i >= lastE)
        def _():
            row = lax.broadcasted_iota(jnp.int32, (BE, 1), 0)
            edge = i * BE + 8 * lax.rem(row, BR) + row // BR
            out_r[...] = repack(jnp.where(edge < E, msg, 0.0))

        @pl.when(i < lastE)
        def _():
            out_r[...] = repack(msg)

    ck_spec = pl.BlockSpec((BR, 128), lambda i: (jnp.minimum(i, lastE), 0))
    pk_spec = lambda d: pl.BlockSpec((BR, d), lambda i: (i, 0))
    full2 = lambda a, b: pl.BlockSpec((a, b), lambda i: (0, 0))

    return pl.pallas_call(
        body,
        grid=(n,),
        in_specs=[
            ck_spec, pk_spec(128), pk_spec(128), ck_spec,
            full2(NSF, NSF), full2(NSF, NSF), full2(NSF, NSF),
            full2(8, NSF), full2(NSF, OUTP), full2(8, OUTP),
            full2(NSF, RW), full2(NSF, RW), full2(RW, OUTP),
        ],
        out_specs=pk_spec(8 * OUTP),
        out_shape=jax.ShapeDtypeStruct((RP, 8 * OUTP), _f32),
    )(eaP, lgP, rgP, shP, w1a, w1b, w1c, b1, w2, b2, wtpall, repmat, summat)


def _tc_combine(parts, cmpmat):
    NPK = NNODE // 8
    partsP = parts.reshape(NCORE, NPK, 8 * OUTP)

    def body(a_r, b_r, c_r, o32_r, o16_r):
        s = a_r[0] + b_r[0]
        o32_r[...] = s
        o16_r[...] = _dot(s, c_r[...])

    return pl.pallas_call(
        body,
        grid=(1,),
        in_specs=[
            pl.BlockSpec((1, NPK, 8 * OUTP), lambda i: (0, 0, 0)),
            pl.BlockSpec((1, NPK, 8 * OUTP), lambda i: (1, 0, 0)),
            pl.BlockSpec((8 * OUTP, 8 * NSF), lambda i: (0, 0)),
        ],
        out_specs=[
            pl.BlockSpec((NPK, 8 * OUTP), lambda i: (0, 0)),
            pl.BlockSpec((NPK, 8 * NSF), lambda i: (0, 0)),
        ],
        out_shape=[
            jax.ShapeDtypeStruct((NPK, 8 * OUTP), _f32),
            jax.ShapeDtypeStruct((NPK, 8 * NSF), _f32),
        ],
    )(partsP, partsP, cmpmat)


def _prep_params(W1, b1, W2, b2, Wtp):
    w1a, w1b, w1c = W1[:NSF], W1[NSF:2 * NSF], W1[2 * NSF:]
    b1b = jnp.broadcast_to(b1[None, :], (8, NSF))
    w2p = jnp.pad(W2, ((0, 0), (0, OUTP - OUTR)))
    b2b = jnp.broadcast_to(jnp.pad(b2, (0, OUTP - OUTR))[None, :], (8, OUTP))
    wtpall = Wtp.reshape(NSF, SHD * OUTR)
    return w1a, w1b, w1c, b1b, w2p, b2b, wtpall


def _const_mats():
    rw = SHD * OUTR
    repmat = (jnp.arange(rw)[None, :] // OUTR
              == jnp.arange(NSF)[:, None]).astype(_f32)
    summat = (jnp.arange(rw)[:, None] % OUTR
              == jnp.arange(OUTP)[None, :]).astype(_f32)
    a = jnp.arange(8 * OUTP)[:, None]
    b = jnp.arange(8 * NSF)[None, :]
    cmpmat = ((a // OUTP == b // NSF) & (a % OUTP == b % NSF)).astype(_f32)
    return repmat, summat, cmpmat


def kernel(lig_node_attr, rec_node_attr, lr_edge_attr, lr_edge_sh,
           W1_0, b1_0, W2_0, b2_0, Wtp_0,
           W1_1, b1_1, W2_1, b2_1, Wtp_1,
           lr_edge_index):
    E = lr_edge_attr.shape[0]
    step = NW * CH * 128
    EP = ((E + step - 1) // step) * step
    pad = EP - E

    src = lr_edge_index[0].astype(jnp.int32)
    dst = lr_edge_index[1].astype(jnp.int32)
    src2d = jnp.concatenate([src, jnp.zeros((pad,), jnp.int32)]
                            ).reshape(EP // 128, 128)
    dst2d = jnp.concatenate([dst, jnp.zeros((pad,), jnp.int32)]
                            ).reshape(EP // 128, 128)
    zrows = jnp.zeros((NNODE // NSUB, OUTP), _f32)
    repmat, summat, cmpmat = _const_mats()
    eaP = lr_edge_attr.reshape(E // 8, 128)
    shP = jnp.pad(lr_edge_sh, ((0, 0), (0, NSF - SHD))).reshape(E // 8, 128)

    rec_g, lig_g = _sc_gather([rec_node_attr, lig_node_attr], [dst2d, src2d])
    rec_gP = rec_g.reshape(EP // 8, 128)
    lig_gP = lig_g.reshape(EP // 8, 128)

    msg0 = _tc_edge(eaP, lig_gP, rec_gP, shP,
                    *_prep_params(W1_0, b1_0, W2_0, b2_0, Wtp_0),
                    repmat, summat)
    parts0 = _sc_scatter_add(msg0.reshape(EP, OUTP), src2d, zrows)
    out0_32P, out0_16P = _tc_combine(parts0, cmpmat)

    lig_g1, = _sc_gather([out0_16P.reshape(NNODE, NSF)], [src2d])
    msg1 = _tc_edge(eaP, lig_g1.reshape(EP // 8, 128), rec_gP, shP,
                    *_prep_params(W1_1, b1_1, W2_1, b2_1, Wtp_1),
                    repmat, summat)
    parts1 = _sc_scatter_add(msg1.reshape(EP, OUTP), src2d, zrows)
    out1_32P, _ = _tc_combine(parts1, cmpmat)
    return out1_32P.reshape(NNODE, OUTP)[:, :OUTR]


# speedup vs baseline: 4.0547x; 1.2441x over previous
import functools

import jax
import jax.numpy as jnp
from jax import lax
from jax.experimental import pallas as pl
from jax.experimental.pallas import tpu as pltpu
from jax.experimental.pallas import tpu_sc as plsc

NSF = 16
SHD = 9
OUTR = 28
OUTP = 32
NNODE = 50000
NCORE = 2
NSUB = 16
NW = NCORE * NSUB
CH = 7
BE = 2048

_f32 = jnp.float32


def _mesh():
    return plsc.VectorSubcoreMesh(core_axis_name="c", subcore_axis_name="s")


_SC_PARAMS = pltpu.CompilerParams(use_tc_tiling_on_sc=False)


def _sc_gather(tables, idx2ds):
    npair = len(tables)
    R = idx2ds[0].shape[0]
    rpw = R // NW
    ngrp = rpw // CH
    half = ngrp // 2
    D = tables[0].shape[1]

    scratch = ([pltpu.VMEM((rpw, 128), jnp.int32) for _ in range(npair)]
               + [pltpu.VMEM((CH * 128, D), _f32) for _ in range(2 * npair)]
               + [pltpu.SemaphoreType.DMA, pltpu.SemaphoreType.DMA,
                  pltpu.SemaphoreType.DMA])

    @functools.partial(
        pl.kernel,
        out_type=[jax.ShapeDtypeStruct((R * 128, D), _f32)
                  for _ in range(npair)],
        mesh=_mesh(),
        scratch_types=scratch,
        compiler_params=_SC_PARAMS,
    )
    def gk(*refs):
        table_h = refs[:npair]
        idx_h = refs[npair:2 * npair]
        out_h = refs[2 * npair:3 * npair]
        idx_v = refs[3 * npair:4 * npair]
        bank = refs[4 * npair:6 * npair]
        gsem, wsem0, wsem1 = refs[6 * npair:]
        wid = lax.axis_index("s") * NCORE + lax.axis_index("c")
        base = wid * rpw

        for p in range(npair):
            pltpu.sync_copy(idx_h[p].at[pl.ds(base, rpw)], idx_v[p])

        def fire_gather(g, b):
            for p in range(npair):
                for j in range(CH):
                    pltpu.async_copy(
                        table_h[p].at[idx_v[p].at[g * CH + j]],
                        bank[b * npair + p].at[pl.ds(j * 128, 128)], gsem)

        def wait_gather():
            for p in range(npair):
                for j in range(CH):
                    pltpu.make_async_copy(
                        table_h[p].at[idx_v[p].at[j]],
                        bank[p].at[pl.ds(j * 128, 128)], gsem).wait()

        def fire_wb(g, b, sem):
            for p in range(npair):
                pltpu.async_copy(
                    bank[b * npair + p],
                    out_h[p].at[pl.ds((base + g * CH) * 128, CH * 128)], sem)

        def wait_wb(sem):
            for p in range(npair):
                pltpu.make_async_copy(
                    bank[p], out_h[p].at[pl.ds(base * 128, CH * 128)],
                    sem).wait()

        fire_gather(0, 0)

        @pl.loop(0, half)
        def _(t):
            wait_gather()
            fire_wb(2 * t, 0, wsem0)

            @pl.when(t > 0)
            def _():
                wait_wb(wsem1)
            fire_gather(2 * t + 1, 1)
            wait_gather()
            wait_wb(wsem0)

            @pl.when(t < half - 1)
            def _():
                fire_gather(2 * t + 2, 0)
            fire_wb(2 * t + 1, 1, wsem1)

        wait_wb(wsem1)

    outs = gk(*tables, *idx2ds)
    return list(outs) if isinstance(outs, (list, tuple)) else [outs]


def _sc_scatter_add(msg, idx2d, zrows):
    R = idx2d.shape[0]
    rpw = R // NW
    SCH = 2
    ngrp = rpw // SCH
    half = ngrp // 2
    rps = NNODE // NSUB

    @functools.partial(
        pl.kernel,
        out_type=jax.ShapeDtypeStruct((NCORE * NNODE, OUTP), _f32),
        mesh=_mesh(),
        scratch_types=[
            pltpu.VMEM((SCH, 128), jnp.int32),
            pltpu.VMEM((SCH, 128), jnp.int32),
            pltpu.VMEM((SCH * 128, OUTP), _f32),
            pltpu.VMEM((SCH * 128, OUTP), _f32),
            pltpu.VMEM_SHARED((NNODE, OUTP), _f32),
            pltpu.SemaphoreType.DMA,
            pltpu.SemaphoreType.DMA,
            pltpu.SemaphoreType.DMA,
        ],
        compiler_params=_SC_PARAMS,
    )
    def sk(msg_hbm, idx_hbm, z_hbm, out_hbm, ib0, ib1, mb0, mb1, acc,
           lsem0, lsem1, ssem):
        cid = lax.axis_index("c")
        sid = lax.axis_index("s")
        wid = sid * NCORE + cid
        base = wid * rpw
        pltpu.sync_copy(z_hbm, acc.at[pl.ds(sid * rps, rps)])
        plsc.subcore_barrier()

        def fire_load(g, ib, mb, sem):
            pltpu.async_copy(idx_hbm.at[pl.ds(base + g * SCH, SCH)], ib, sem)
            pltpu.async_copy(
                msg_hbm.at[pl.ds((base + g * SCH) * 128, SCH * 128)], mb, sem)

        def wait_load(ib, mb, sem):
            pltpu.make_async_copy(
                idx_hbm.at[pl.ds(base, SCH)], ib, sem).wait()
            pltpu.make_async_copy(
                msg_hbm.at[pl.ds(base * 128, SCH * 128)], mb, sem).wait()

        def scatter_group(ib, mb):
            descs = []
            for j in range(SCH):
                descs.append(pltpu.async_copy(
                    mb.at[pl.ds(j * 128, 128)],
                    acc.at[ib.at[j]], ssem, add=True))
            for d in descs:
                d.wait()

        fire_load(0, ib0, mb0, lsem0)

        @pl.loop(0, half)
        def _(t):
            wait_load(ib0, mb0, lsem0)
            fire_load(2 * t + 1, ib1, mb1, lsem1)
            scatter_group(ib0, mb0)
            wait_load(ib1, mb1, lsem1)

            @pl.when(t < half - 1)
            def _():
                fire_load(2 * t + 2, ib0, mb0, lsem0)
            scatter_group(ib1, mb1)

        plsc.subcore_barrier()
        pltpu.sync_copy(acc.at[pl.ds(sid * rps, rps)],
                        out_hbm.at[pl.ds(cid * NNODE + sid * rps, rps)])

    return sk(msg, idx2d, zrows)


def _dot(a, b):
    return jax.lax.dot_general(a, b, (((1,), (0,)), ((), ())),
                               preferred_element_type=_f32)


def _dot_c0(a, b):
    return jax.lax.dot_general(a, b, (((0,), (0,)), ((), ())),
                               preferred_element_type=_f32)


def _tc_edge(eaP, lgP, rgP, shP, w1a, w1b, w1c, b1, w2, b2, wtpall, repmat,
             summat):
    E = eaP.shape[0] * 8
    RP = lgP.shape[0]
    BR = BE // 8
    n = RP // BR
    lastE = (E // 8 - 1) // BR
    RW8 = 8 * SHD * OUTR

    def body(ea_r, lg_r, rg_r, sh_r, w1a_r, w1b_r, w1c_r, b1_r, w2_r,
             b2_r, wtp_r, rep_r, sum_r, out_r):
        i = pl.program_id(0)
        rgp = rg_r[...]
        hp = jnp.maximum(
            _dot(ea_r[...], w1a_r[...]) + _dot(lg_r[...], w1b_r[...])
            + _dot(rgp, w1c_r[...]) + b1_r[0:1, :], 0.0)
        gatep = _dot(hp, w2_r[...]) + b2_r[0:1, :]
        up = _dot(rgp, wtp_r[...])
        sp = _dot(sh_r[...], rep_r[...])
        msgp = _dot(up * sp, sum_r[...]) * gatep

        @pl.when(i >= lastE)
        def _():
            row = lax.broadcasted_iota(jnp.int32, (BR, 8 * OUTP), 0)
            lane = lax.broadcasted_iota(jnp.int32, (BR, 8 * OUTP), 1)
            edge = i * BE + 8 * row + lane // OUTP
            out_r[...] = jnp.where(edge < E, msgp, 0.0)

        @pl.when(i < lastE)
        def _():
            out_r[...] = msgp

    ck_spec = pl.BlockSpec((BR, 128), lambda i: (jnp.minimum(i, lastE), 0))
    pk_spec = lambda d: pl.BlockSpec((BR, d), lambda i: (i, 0))
    full2 = lambda a, b: pl.BlockSpec((a, b), lambda i: (0, 0))

    return pl.pallas_call(
        body,
        grid=(n,),
        in_specs=[
            ck_spec, pk_spec(128), pk_spec(128), ck_spec,
            full2(128, 128), full2(128, 128), full2(128, 128),
            full2(8, 128), full2(128, 8 * OUTP), full2(8, 8 * OUTP),
            full2(128, RW8), full2(128, RW8), full2(RW8, 8 * OUTP),
        ],
        out_specs=pk_spec(8 * OUTP),
        out_shape=jax.ShapeDtypeStruct((RP, 8 * OUTP), _f32),
    )(eaP, lgP, rgP, shP, w1a, w1b, w1c, b1, w2, b2, wtpall, repmat, summat)


def _tc_combine(parts, cmpmat):
    NPK = NNODE // 8
    partsP = parts.reshape(NCORE, NPK, 8 * OUTP)

    def body(a_r, b_r, c_r, o32_r, o16_r):
        s = a_r[0] + b_r[0]
        o32_r[...] = s
        o16_r[...] = _dot(s, c_r[...])

    return pl.pallas_call(
        body,
        grid=(1,),
        in_specs=[
            pl.BlockSpec((1, NPK, 8 * OUTP), lambda i: (0, 0, 0)),
            pl.BlockSpec((1, NPK, 8 * OUTP), lambda i: (1, 0, 0)),
            pl.BlockSpec((8 * OUTP, 8 * NSF), lambda i: (0, 0)),
        ],
        out_specs=[
            pl.BlockSpec((NPK, 8 * OUTP), lambda i: (0, 0)),
            pl.BlockSpec((NPK, 8 * NSF), lambda i: (0, 0)),
        ],
        out_shape=[
            jax.ShapeDtypeStruct((NPK, 8 * OUTP), _f32),
            jax.ShapeDtypeStruct((NPK, 8 * NSF), _f32),
        ],
    )(partsP, partsP, cmpmat)


def _bd8(w):
    return jnp.kron(jnp.eye(8, dtype=_f32), w)


def _prep_params(W1, b1, W2, b2, Wtp):
    w1a, w1b, w1c = (_bd8(W1[:NSF]), _bd8(W1[NSF:2 * NSF]),
                     _bd8(W1[2 * NSF:]))
    b1b = jnp.broadcast_to(jnp.tile(b1, 8)[None, :], (8, 128))
    w2p = _bd8(jnp.pad(W2, ((0, 0), (0, OUTP - OUTR))))
    b2b = jnp.broadcast_to(
        jnp.tile(jnp.pad(b2, (0, OUTP - OUTR)), 8)[None, :], (8, 8 * OUTP))
    wtpall = _bd8(Wtp.reshape(NSF, SHD * OUTR))
    return w1a, w1b, w1c, b1b, w2p, b2b, wtpall


def _const_mats():
    rw = SHD * OUTR
    repmat = _bd8((jnp.arange(rw)[None, :] // OUTR
                   == jnp.arange(NSF)[:, None]).astype(_f32))
    summat = _bd8((jnp.arange(rw)[:, None] % OUTR
                   == jnp.arange(OUTP)[None, :]).astype(_f32))
    a = jnp.arange(8 * OUTP)[:, None]
    b = jnp.arange(8 * NSF)[None, :]
    cmpmat = ((a // OUTP == b // NSF) & (a % OUTP == b % NSF)).astype(_f32)
    return repmat, summat, cmpmat


def kernel(lig_node_attr, rec_node_attr, lr_edge_attr, lr_edge_sh,
           W1_0, b1_0, W2_0, b2_0, Wtp_0,
           W1_1, b1_1, W2_1, b2_1, Wtp_1,
           lr_edge_index):
    E = lr_edge_attr.shape[0]
    step = NW * CH * 128
    EP = ((E + step - 1) // step) * step
    pad = EP - E

    src = lr_edge_index[0].astype(jnp.int32)
    dst = lr_edge_index[1].astype(jnp.int32)
    src2d = jnp.concatenate([src, jnp.zeros((pad,), jnp.int32)]
                            ).reshape(EP // 128, 128)
    dst2d = jnp.concatenate([dst, jnp.zeros((pad,), jnp.int32)]
                            ).reshape(EP // 128, 128)
    zrows = jnp.zeros((NNODE // NSUB, OUTP), _f32)
    repmat, summat, cmpmat = _const_mats()
    eaP = lr_edge_attr.reshape(E // 8, 128)
    shP = jnp.pad(lr_edge_sh, ((0, 0), (0, NSF - SHD))).reshape(E // 8, 128)

    rec_g, lig_g = _sc_gather([rec_node_attr, lig_node_attr], [dst2d, src2d])
    rec_gP = rec_g.reshape(EP // 8, 128)
    lig_gP = lig_g.reshape(EP // 8, 128)

    msg0 = _tc_edge(eaP, lig_gP, rec_gP, shP,
                    *_prep_params(W1_0, b1_0, W2_0, b2_0, Wtp_0),
                    repmat, summat)
    parts0 = _sc_scatter_add(msg0.reshape(EP, OUTP), src2d, zrows)
    out0_32P, out0_16P = _tc_combine(parts0, cmpmat)

    lig_g1, = _sc_gather([out0_16P.reshape(NNODE, NSF)], [src2d])
    msg1 = _tc_edge(eaP, lig_g1.reshape(EP // 8, 128), rec_gP, shP,
                    *_prep_params(W1_1, b1_1, W2_1, b2_1, Wtp_1),
                    repmat, summat)
    parts1 = _sc_scatter_add(msg1.reshape(EP, OUTP), src2d, zrows)
    out1_32P, _ = _tc_combine(parts1, cmpmat)
    return out1_32P.reshape(NNODE, OUTP)[:, :OUTR]


# speedup vs baseline: 4.3906x; 1.0828x over previous
import functools

import jax
import jax.numpy as jnp
from jax import lax
from jax.experimental import pallas as pl
from jax.experimental.pallas import tpu as pltpu
from jax.experimental.pallas import tpu_sc as plsc

NSF = 16
SHD = 9
OUTR = 28
OUTP = 32
NNODE = 50000
NCORE = 2
NSUB = 16
NW = NCORE * NSUB
CH = 7
BE = 4096

_f32 = jnp.float32


def _mesh():
    return plsc.VectorSubcoreMesh(core_axis_name="c", subcore_axis_name="s")


_SC_PARAMS = pltpu.CompilerParams(use_tc_tiling_on_sc=False)


def _sc_gather(tables, idx2ds):
    npair = len(tables)
    R = idx2ds[0].shape[0]
    rpw = R // NW
    ngrp = rpw // CH
    half = ngrp // 2
    D = tables[0].shape[1]

    scratch = ([pltpu.VMEM((rpw, 128), jnp.int32) for _ in range(npair)]
               + [pltpu.VMEM((CH * 128, D), _f32) for _ in range(2 * npair)]
               + [pltpu.SemaphoreType.DMA, pltpu.SemaphoreType.DMA,
                  pltpu.SemaphoreType.DMA])

    @functools.partial(
        pl.kernel,
        out_type=[jax.ShapeDtypeStruct((R * 128, D), _f32)
                  for _ in range(npair)],
        mesh=_mesh(),
        scratch_types=scratch,
        compiler_params=_SC_PARAMS,
    )
    def gk(*refs):
        table_h = refs[:npair]
        idx_h = refs[npair:2 * npair]
        out_h = refs[2 * npair:3 * npair]
        idx_v = refs[3 * npair:4 * npair]
        bank = refs[4 * npair:6 * npair]
        gsem, wsem0, wsem1 = refs[6 * npair:]
        wid = lax.axis_index("s") * NCORE + lax.axis_index("c")
        base = wid * rpw

        for p in range(npair):
            pltpu.sync_copy(idx_h[p].at[pl.ds(base, rpw)], idx_v[p])

        def fire_gather(g, b):
            for p in range(npair):
                for j in range(CH):
                    pltpu.async_copy(
                        table_h[p].at[idx_v[p].at[g * CH + j]],
                        bank[b * npair + p].at[pl.ds(j * 128, 128)], gsem)

        def wait_gather():
            for p in range(npair):
                for j in range(CH):
                    pltpu.make_async_copy(
                        table_h[p].at[idx_v[p].at[j]],
                        bank[p].at[pl.ds(j * 128, 128)], gsem).wait()

        def fire_wb(g, b, sem):
            for p in range(npair):
                pltpu.async_copy(
                    bank[b * npair + p],
                    out_h[p].at[pl.ds((base + g * CH) * 128, CH * 128)], sem)

        def wait_wb(sem):
            for p in range(npair):
                pltpu.make_async_copy(
                    bank[p], out_h[p].at[pl.ds(base * 128, CH * 128)],
                    sem).wait()

        fire_gather(0, 0)

        @pl.loop(0, half)
        def _(t):
            wait_gather()
            fire_wb(2 * t, 0, wsem0)

            @pl.when(t > 0)
            def _():
                wait_wb(wsem1)
            fire_gather(2 * t + 1, 1)
            wait_gather()
            wait_wb(wsem0)

            @pl.when(t < half - 1)
            def _():
                fire_gather(2 * t + 2, 0)
            fire_wb(2 * t + 1, 1, wsem1)

        wait_wb(wsem1)

    outs = gk(*tables, *idx2ds)
    return list(outs) if isinstance(outs, (list, tuple)) else [outs]


def _sc_scatter_add(msg, idx2d, zrows):
    R = idx2d.shape[0]
    rpw = R // NW
    SCH = 2
    ngrp = rpw // SCH
    half = ngrp // 2
    rps = NNODE // NSUB

    @functools.partial(
        pl.kernel,
        out_type=jax.ShapeDtypeStruct((NCORE * NNODE, OUTP), _f32),
        mesh=_mesh(),
        scratch_types=[
            pltpu.VMEM((SCH, 128), jnp.int32),
            pltpu.VMEM((SCH, 128), jnp.int32),
            pltpu.VMEM((SCH * 128, OUTP), _f32),
            pltpu.VMEM((SCH * 128, OUTP), _f32),
            pltpu.VMEM_SHARED((NNODE, OUTP), _f32),
            pltpu.SemaphoreType.DMA,
            pltpu.SemaphoreType.DMA,
            pltpu.SemaphoreType.DMA,
        ],
        compiler_params=_SC_PARAMS,
    )
    def sk(msg_hbm, idx_hbm, z_hbm, out_hbm, ib0, ib1, mb0, mb1, acc,
           lsem0, lsem1, ssem):
        cid = lax.axis_index("c")
        sid = lax.axis_index("s")
        wid = sid * NCORE + cid
        base = wid * rpw
        pltpu.sync_copy(z_hbm, acc.at[pl.ds(sid * rps, rps)])
        plsc.subcore_barrier()

        def fire_load(g, ib, mb, sem):
            pltpu.async_copy(idx_hbm.at[pl.ds(base + g * SCH, SCH)], ib, sem)
            pltpu.async_copy(
                msg_hbm.at[pl.ds((base + g * SCH) * 128, SCH * 128)], mb, sem)

        def wait_load(ib, mb, sem):
            pltpu.make_async_copy(
                idx_hbm.at[pl.ds(base, SCH)], ib, sem).wait()
            pltpu.make_async_copy(
                msg_hbm.at[pl.ds(base * 128, SCH * 128)], mb, sem).wait()

        def scatter_group(ib, mb):
            descs = []
            for j in range(SCH):
                descs.append(pltpu.async_copy(
                    mb.at[pl.ds(j * 128, 128)],
                    acc.at[ib.at[j]], ssem, add=True))
            for d in descs:
                d.wait()

        fire_load(0, ib0, mb0, lsem0)

        @pl.loop(0, half)
        def _(t):
            wait_load(ib0, mb0, lsem0)
            fire_load(2 * t + 1, ib1, mb1, lsem1)
            scatter_group(ib0, mb0)
            wait_load(ib1, mb1, lsem1)

            @pl.when(t < half - 1)
            def _():
                fire_load(2 * t + 2, ib0, mb0, lsem0)
            scatter_group(ib1, mb1)

        plsc.subcore_barrier()
        pltpu.sync_copy(acc.at[pl.ds(sid * rps, rps)],
                        out_hbm.at[pl.ds(cid * NNODE + sid * rps, rps)])

    return sk(msg, idx2d, zrows)


def _dot(a, b):
    return jax.lax.dot_general(a, b, (((1,), (0,)), ((), ())),
                               preferred_element_type=_f32)


def _dot_c0(a, b):
    return jax.lax.dot_general(a, b, (((0,), (0,)), ((), ())),
                               preferred_element_type=_f32)


def _tc_edge(eaP, lgP, rgP, shP, w1a, w1b, w1c, b1, w2, b2, wtpall, repmat,
             summat):
    E = eaP.shape[0] * 8
    RP = lgP.shape[0]
    BR = BE // 8
    n = RP // BR
    lastE = (E // 8 - 1) // BR
    RW8 = 8 * SHD * OUTR

    def body(ea_r, lg_r, rg_r, sh_r, w1a_r, w1b_r, w1c_r, b1_r, w2_r,
             b2_r, wtp_r, rep_r, sum_r, out_r):
        i = pl.program_id(0)
        rgp = rg_r[...]
        hp = jnp.maximum(
            _dot(ea_r[...], w1a_r[...]) + _dot(lg_r[...], w1b_r[...])
            + _dot(rgp, w1c_r[...]) + b1_r[0:1, :], 0.0)
        gatep = _dot(hp, w2_r[...]) + b2_r[0:1, :]
        up = _dot(rgp, wtp_r[...])
        sp = _dot(sh_r[...], rep_r[...])
        msgp = _dot(up * sp, sum_r[...]) * gatep

        @pl.when(i >= lastE)
        def _():
            row = lax.broadcasted_iota(jnp.int32, (BR, 8 * OUTP), 0)
            lane = lax.broadcasted_iota(jnp.int32, (BR, 8 * OUTP), 1)
            edge = i * BE + 8 * row + lane // OUTP
            out_r[...] = jnp.where(edge < E, msgp, 0.0)

        @pl.when(i < lastE)
        def _():
            out_r[...] = msgp

    ck_spec = pl.BlockSpec((BR, 128), lambda i: (jnp.minimum(i, lastE), 0))
    pk_spec = lambda d: pl.BlockSpec((BR, d), lambda i: (i, 0))
    full2 = lambda a, b: pl.BlockSpec((a, b), lambda i: (0, 0))

    return pl.pallas_call(
        body,
        grid=(n,),
        in_specs=[
            ck_spec, pk_spec(128), pk_spec(128), ck_spec,
            full2(128, 128), full2(128, 128), full2(128, 128),
            full2(8, 128), full2(128, 8 * OUTP), full2(8, 8 * OUTP),
            full2(128, RW8), full2(128, RW8), full2(RW8, 8 * OUTP),
        ],
        out_specs=pk_spec(8 * OUTP),
        out_shape=jax.ShapeDtypeStruct((RP, 8 * OUTP), _f32),
    )(eaP, lgP, rgP, shP, w1a, w1b, w1c, b1, w2, b2, wtpall, repmat, summat)


def _tc_combine(parts, cmpmat):
    NPK = NNODE // 8
    partsP = parts.reshape(NCORE, NPK, 8 * OUTP)

    def body(a_r, b_r, c_r, o32_r, o16_r):
        s = a_r[0] + b_r[0]
        o32_r[...] = s
        o16_r[...] = _dot(s, c_r[...])

    return pl.pallas_call(
        body,
        grid=(1,),
        in_specs=[
            pl.BlockSpec((1, NPK, 8 * OUTP), lambda i: (0, 0, 0)),
            pl.BlockSpec((1, NPK, 8 * OUTP), lambda i: (1, 0, 0)),
            pl.BlockSpec((8 * OUTP, 8 * NSF), lambda i: (0, 0)),
        ],
        out_specs=[
            pl.BlockSpec((NPK, 8 * OUTP), lambda i: (0, 0)),
            pl.BlockSpec((NPK, 8 * NSF), lambda i: (0, 0)),
        ],
        out_shape=[
            jax.ShapeDtypeStruct((NPK, 8 * OUTP), _f32),
            jax.ShapeDtypeStruct((NPK, 8 * NSF), _f32),
        ],
    )(partsP, partsP, cmpmat)


def _bd8(w):
    return jnp.kron(jnp.eye(8, dtype=_f32), w)


def _prep_params(W1, b1, W2, b2, Wtp):
    w1a, w1b, w1c = (_bd8(W1[:NSF]), _bd8(W1[NSF:2 * NSF]),
                     _bd8(W1[2 * NSF:]))
    b1b = jnp.broadcast_to(jnp.tile(b1, 8)[None, :], (8, 128))
    w2p = _bd8(jnp.pad(W2, ((0, 0), (0, OUTP - OUTR))))
    b2b = jnp.broadcast_to(
        jnp.tile(jnp.pad(b2, (0, OUTP - OUTR)), 8)[None, :], (8, 8 * OUTP))
    wtpall = _bd8(Wtp.reshape(NSF, SHD * OUTR))
    return w1a, w1b, w1c, b1b, w2p, b2b, wtpall


def _const_mats():
    rw = SHD * OUTR
    repmat = _bd8((jnp.arange(rw)[None, :] // OUTR
                   == jnp.arange(NSF)[:, None]).astype(_f32))
    summat = _bd8((jnp.arange(rw)[:, None] % OUTR
                   == jnp.arange(OUTP)[None, :]).astype(_f32))
    a = jnp.arange(8 * OUTP)[:, None]
    b = jnp.arange(8 * NSF)[None, :]
    cmpmat = ((a // OUTP == b // NSF) & (a % OUTP == b % NSF)).astype(_f32)
    return repmat, summat, cmpmat


def kernel(lig_node_attr, rec_node_attr, lr_edge_attr, lr_edge_sh,
           W1_0, b1_0, W2_0, b2_0, Wtp_0,
           W1_1, b1_1, W2_1, b2_1, Wtp_1,
           lr_edge_index):
    E = lr_edge_attr.shape[0]
    step = NW * CH * 128
    EP = ((E + step - 1) // step) * step
    pad = EP - E

    src = lr_edge_index[0].astype(jnp.int32)
    dst = lr_edge_index[1].astype(jnp.int32)
    src2d = jnp.concatenate([src, jnp.zeros((pad,), jnp.int32)]
                            ).reshape(EP // 128, 128)
    dst2d = jnp.concatenate([dst, jnp.zeros((pad,), jnp.int32)]
                            ).reshape(EP // 128, 128)
    zrows = jnp.zeros((NNODE // NSUB, OUTP), _f32)
    repmat, summat, cmpmat = _const_mats()
    eaP = lr_edge_attr.reshape(E // 8, 128)
    shP = jnp.pad(lr_edge_sh, ((0, 0), (0, NSF - SHD))).reshape(E // 8, 128)

    rec_g, lig_g = _sc_gather([rec_node_attr, lig_node_attr], [dst2d, src2d])
    rec_gP = rec_g.reshape(EP // 8, 128)
    lig_gP = lig_g.reshape(EP // 8, 128)

    msg0 = _tc_edge(eaP, lig_gP, rec_gP, shP,
                    *_prep_params(W1_0, b1_0, W2_0, b2_0, Wtp_0),
                    repmat, summat)
    parts0 = _sc_scatter_add(msg0.reshape(EP, OUTP), src2d, zrows)
    out0_32P, out0_16P = _tc_combine(parts0, cmpmat)

    lig_g1, = _sc_gather([out0_16P.reshape(NNODE, NSF)], [src2d])
    msg1 = _tc_edge(eaP, lig_g1.reshape(EP // 8, 128), rec_gP, shP,
                    *_prep_params(W1_1, b1_1, W2_1, b2_1, Wtp_1),
                    repmat, summat)
    parts1 = _sc_scatter_add(msg1.reshape(EP, OUTP), src2d, zrows)
    out1_32P, _ = _tc_combine(parts1, cmpmat)
    return out1_32P.reshape(NNODE, OUTP)[:, :OUTR]
